# Initial kernel scaffold; baseline (speedup 1.0000x reference)
import functools

import jax
import jax.numpy as jnp
from jax import lax
from jax.experimental import pallas as pl
from jax.experimental.pallas import tpu as pltpu
from jax.experimental.pallas import tpu_sc as plsc

N = 10000
D_IN = 128
F1 = 48
F2 = 60
F2P = 64

NC = 2
NS = 16
NW = NC * NS

NPAD = 10240
RPS = NPAD // NS

EB = 128
NB = 80
EPT = NB * EB
EPAD = NW * EPT

ROWB = 256
GRID = NPAD // ROWB

_MESH = plsc.VectorSubcoreMesh(core_axis_name="c", subcore_axis_name="s")


@functools.partial(
    pl.kernel,
    out_type=jax.ShapeDtypeStruct((NC, NPAD), jnp.float32),
    mesh=_MESH,
    scratch_types=[
        pltpu.VMEM((NB, EB), jnp.int32),
        pltpu.VMEM((NB, EB), jnp.float32),
        pltpu.VMEM((RPS,), jnp.float32),
        pltpu.VMEM_SHARED((NPAD,), jnp.float32),
    ],
)
def _deg_kernel(dst_hbm, w_hbm, out_hbm, dst_v, w_v, zbuf, deg_sh):
    cid = lax.axis_index("c")
    sid = lax.axis_index("s")
    wid = cid * NS + sid
    pltpu.sync_copy(dst_hbm.at[wid], dst_v)
    pltpu.sync_copy(w_hbm.at[wid], w_v)

    zero16 = jnp.zeros((16,), jnp.float32)

    @pl.loop(0, RPS // 16)
    def _(i):
        zbuf[pl.ds(i * 16, 16)] = zero16

    pltpu.sync_copy(zbuf, deg_sh.at[pl.ds(sid * RPS, RPS)])
    plsc.subcore_barrier()

    @pl.loop(0, NB)
    def _(j):
        pltpu.sync_copy(w_v.at[j], deg_sh.at[dst_v.at[j]], add=True)

    plsc.subcore_barrier()
    pltpu.sync_copy(deg_sh.at[pl.ds(sid * RPS, RPS)],
                    out_hbm.at[cid, pl.ds(sid * RPS, RPS)])


def _make_agg(fp):
    nk = fp // 16

    @functools.partial(
        pl.kernel,
        out_type=jax.ShapeDtypeStruct((NC, NPAD, fp), jnp.float32),
        mesh=_MESH,
        scratch_types=[
            pltpu.VMEM((NB, EB), jnp.int32),
            pltpu.VMEM((NB, EB), jnp.int32),
            pltpu.VMEM((NB, EB), jnp.float32),
            pltpu.VMEM((EB, fp), jnp.float32),
            pltpu.VMEM((EB, fp), jnp.float32),
            pltpu.VMEM_SHARED((NPAD, fp), jnp.float32),
            pltpu.SemaphoreType.DMA,
            pltpu.SemaphoreType.DMA,
        ],
    )
    def agg_kernel(h_hbm, src_hbm, dst_hbm, w_hbm, out_hbm,
                   src_v, dst_v, w_v, rows0, rows1, acc_sh, sem0, sem1):
        cid = lax.axis_index("c")
        sid = lax.axis_index("s")
        wid = cid * NS + sid
        pltpu.sync_copy(src_hbm.at[wid], src_v)
        pltpu.sync_copy(dst_hbm.at[wid], dst_v)
        pltpu.sync_copy(w_hbm.at[wid], w_v)

        zero16 = jnp.zeros((16,), jnp.float32)

        @pl.loop(0, EB)
        def _(r):
            for k in range(nk):
                rows0[r, pl.ds(k * 16, 16)] = zero16

        @pl.loop(0, RPS // EB)
        def _(i):
            pltpu.sync_copy(rows0, acc_sh.at[pl.ds(sid * RPS + i * EB, EB)])

        plsc.subcore_barrier()

        bufs = (rows0, rows1)
        sems = (sem0, sem1)
        pltpu.async_copy(h_hbm.at[src_v.at[0]], rows0, sem0)
        pltpu.async_copy(h_hbm.at[src_v.at[1]], rows1, sem1)

        @pl.loop(0, NB, step=2)
        def _(j0):
            for b in range(2):
                j = j0 + b
                buf, sem = bufs[b], sems[b]
                pltpu.make_async_copy(h_hbm.at[src_v.at[j]], buf, sem).wait()

                def srow(r, _, buf=buf, j=j):
                    wv = jnp.full((16,), w_v[j, r])
                    for k in range(nk):
                        buf[r, pl.ds(k * 16, 16)] = buf[r, pl.ds(k * 16, 16)] * wv
                    return 0

                lax.fori_loop(0, EB, srow, 0)

                pltpu.sync_copy(buf, acc_sh.at[dst_v.at[j]], add=True)

                @pl.when(j + 2 < NB)
                def _(buf=buf, sem=sem, j=j):
                    pltpu.async_copy(h_hbm.at[src_v.at[j + 2]], buf, sem)

        plsc.subcore_barrier()
        pltpu.sync_copy(acc_sh.at[pl.ds(sid * RPS, RPS)],
                        out_hbm.at[cid, pl.ds(sid * RPS, RPS)])

    return agg_kernel


_agg_f1 = _make_agg(F1)
_agg_f2 = _make_agg(F2P)


def _layer1(x_pad, W0, deg0, deg1):
    def body(x_ref, w_ref, d0_ref, d1_ref, dis_ref, h_ref):
        deg = d0_ref[...] + d1_ref[...] + 1.0
        dis = lax.rsqrt(deg)
        dis_ref[...] = dis
        h = jnp.dot(x_ref[...], w_ref[...], preferred_element_type=jnp.float32)
        h_ref[...] = dis * h

    return pl.pallas_call(
        body,
        grid=(GRID,),
        in_specs=[
            pl.BlockSpec((ROWB, D_IN), lambda i: (i, 0)),
            pl.BlockSpec((D_IN, F1), lambda i: (0, 0)),
            pl.BlockSpec((ROWB, 1), lambda i: (i, 0)),
            pl.BlockSpec((ROWB, 1), lambda i: (i, 0)),
        ],
        out_specs=[
            pl.BlockSpec((ROWB, 1), lambda i: (i, 0)),
            pl.BlockSpec((ROWB, F1), lambda i: (i, 0)),
        ],
        out_shape=[
            jax.ShapeDtypeStruct((NPAD, 1), jnp.float32),
            jax.ShapeDtypeStruct((NPAD, F1), jnp.float32),
        ],
    )(x_pad, W0, deg0, deg1)


def _layer2(a0, a1, h1p, dis, W1p):
    def body(a0_ref, a1_ref, h_ref, dis_ref, w_ref, out_ref):
        dis = dis_ref[...]
        t = jnp.maximum(dis * (a0_ref[...] + a1_ref[...] + h_ref[...]), 0.0)
        out_ref[...] = dis * jnp.dot(t, w_ref[...],
                                     preferred_element_type=jnp.float32)

    return pl.pallas_call(
        body,
        grid=(GRID,),
        in_specs=[
            pl.BlockSpec((ROWB, F1), lambda i: (i, 0)),
            pl.BlockSpec((ROWB, F1), lambda i: (i, 0)),
            pl.BlockSpec((ROWB, F1), lambda i: (i, 0)),
            pl.BlockSpec((ROWB, 1), lambda i: (i, 0)),
            pl.BlockSpec((F1, F2P), lambda i: (0, 0)),
        ],
        out_specs=pl.BlockSpec((ROWB, F2P), lambda i: (i, 0)),
        out_shape=jax.ShapeDtypeStruct((NPAD, F2P), jnp.float32),
    )(a0, a1, h1p, dis, W1p)


def _final(a0, a1, h2p, dis):
    def body(a0_ref, a1_ref, h_ref, dis_ref, out_ref):
        dis = dis_ref[...]
        out_ref[...] = jnp.maximum(dis * (a0_ref[...] + a1_ref[...] + h_ref[...]),
                                   0.0)

    return pl.pallas_call(
        body,
        grid=(GRID,),
        in_specs=[
            pl.BlockSpec((ROWB, F2P), lambda i: (i, 0)),
            pl.BlockSpec((ROWB, F2P), lambda i: (i, 0)),
            pl.BlockSpec((ROWB, F2P), lambda i: (i, 0)),
            pl.BlockSpec((ROWB, 1), lambda i: (i, 0)),
        ],
        out_specs=pl.BlockSpec((ROWB, F2P), lambda i: (i, 0)),
        out_shape=jax.ShapeDtypeStruct((NPAD, F2P), jnp.float32),
    )(a0, a1, h2p, dis)


def kernel(x, edge_index, edge_weight, W0, W1):
    src = edge_index[0].astype(jnp.int32)
    dst = edge_index[1].astype(jnp.int32)
    w = edge_weight.astype(jnp.float32)

    pad = EPAD - src.shape[0]
    src_p = jnp.pad(src, (0, pad)).reshape(NW, NB, EB)
    dst_p = jnp.pad(dst, (0, pad)).reshape(NW, NB, EB)
    w_p = jnp.pad(w, (0, pad)).reshape(NW, NB, EB)
    x_pad = jnp.pad(x, ((0, NPAD - N), (0, 0)))
    W1p = jnp.pad(W1, ((0, 0), (0, F2P - F2)))

    deg_p = _deg_kernel(dst_p, w_p)
    deg0 = deg_p[0].reshape(NPAD, 1)
    deg1 = deg_p[1].reshape(NPAD, 1)

    dis, h1p = _layer1(x_pad, W0, deg0, deg1)
    acc1 = _agg_f1(h1p, src_p, dst_p, w_p)
    h2p = _layer2(acc1[0], acc1[1], h1p, dis, W1p)
    acc2 = _agg_f2(h2p, src_p, dst_p, w_p)
    outp = _final(acc2[0], acc2[1], h2p, dis)
    return outp[:N, :F2]


# speedup vs baseline: 17.1528x; 17.1528x over previous
import functools

import jax
import jax.numpy as jnp
from jax import lax
from jax.experimental import pallas as pl
from jax.experimental.pallas import tpu as pltpu
from jax.experimental.pallas import tpu_sc as plsc

N = 10000
D_IN = 128
F1 = 48
F2 = 60
F2P = 64

NC = 2
NS = 16
NW = NC * NS

NPAD = 10240
RPS = NPAD // NS

EB = 128
NB = 80
EPT = NB * EB
EPAD = NW * EPT

ROWB = 256
GRID = NPAD // ROWB

_MESH = plsc.VectorSubcoreMesh(core_axis_name="c", subcore_axis_name="s")


@functools.partial(
    pl.kernel,
    out_type=jax.ShapeDtypeStruct((NC, NPAD), jnp.float32),
    mesh=_MESH,
    compiler_params=pltpu.CompilerParams(use_tc_tiling_on_sc=False),
    scratch_types=[
        pltpu.VMEM((NB, EB), jnp.int32),
        pltpu.VMEM((NB, EB), jnp.float32),
        pltpu.VMEM((RPS,), jnp.float32),
        pltpu.VMEM_SHARED((NPAD,), jnp.float32),
    ],
)
def _deg_kernel(dst_hbm, w_hbm, out_hbm, dst_v, w_v, zbuf, deg_sh):
    cid = lax.axis_index("c")
    sid = lax.axis_index("s")
    wid = cid * NS + sid
    pltpu.sync_copy(dst_hbm.at[wid], dst_v)
    pltpu.sync_copy(w_hbm.at[wid], w_v)

    zero16 = jnp.zeros((16,), jnp.float32)

    @pl.loop(0, RPS // 16)
    def _(i):
        zbuf[pl.ds(i * 16, 16)] = zero16

    pltpu.sync_copy(zbuf, deg_sh.at[pl.ds(sid * RPS, RPS)])
    plsc.subcore_barrier()

    @pl.loop(0, NB)
    def _(j):
        pltpu.sync_copy(w_v.at[j], deg_sh.at[dst_v.at[j]], add=True)

    plsc.subcore_barrier()
    pltpu.sync_copy(deg_sh.at[pl.ds(sid * RPS, RPS)],
                    out_hbm.at[cid, pl.ds(sid * RPS, RPS)])


def _make_agg(fp):
    nk = fp // 16

    @functools.partial(
        pl.kernel,
        out_type=jax.ShapeDtypeStruct((NC, NPAD, fp), jnp.float32),
        mesh=_MESH,
        compiler_params=pltpu.CompilerParams(use_tc_tiling_on_sc=False),
        scratch_types=[
            pltpu.VMEM((NB, EB), jnp.int32),
            pltpu.VMEM((NB, EB), jnp.int32),
            pltpu.VMEM((NB, EB), jnp.float32),
            pltpu.VMEM((EB, fp), jnp.float32),
            pltpu.VMEM((EB, fp), jnp.float32),
            pltpu.VMEM_SHARED((NPAD, fp), jnp.float32),
            pltpu.SemaphoreType.DMA,
            pltpu.SemaphoreType.DMA,
        ],
    )
    def agg_kernel(h_hbm, src_hbm, dst_hbm, w_hbm, out_hbm,
                   src_v, dst_v, w_v, rows0, rows1, acc_sh, sem0, sem1):
        cid = lax.axis_index("c")
        sid = lax.axis_index("s")
        wid = cid * NS + sid
        pltpu.sync_copy(src_hbm.at[wid], src_v)
        pltpu.sync_copy(dst_hbm.at[wid], dst_v)
        pltpu.sync_copy(w_hbm.at[wid], w_v)

        zero16 = jnp.zeros((16,), jnp.float32)

        @pl.loop(0, EB)
        def _(r):
            for k in range(nk):
                rows0[r, pl.ds(k * 16, 16)] = zero16

        @pl.loop(0, RPS // EB)
        def _(i):
            pltpu.sync_copy(rows0, acc_sh.at[pl.ds(sid * RPS + i * EB, EB)])

        plsc.subcore_barrier()

        bufs = (rows0, rows1)
        sems = (sem0, sem1)
        pltpu.async_copy(h_hbm.at[src_v.at[0]], rows0, sem0)
        pltpu.async_copy(h_hbm.at[src_v.at[1]], rows1, sem1)

        @pl.loop(0, NB, step=2)
        def _(j0):
            for b in range(2):
                j = j0 + b
                buf, sem = bufs[b], sems[b]
                pltpu.make_async_copy(h_hbm.at[src_v.at[j]], buf, sem).wait()

                def sgrp(g, _, buf=buf, j=j):
                    w16 = w_v[j, pl.ds(g * 16, 16)]
                    for i in range(16):
                        r = g * 16 + i
                        wv = jnp.full((16,), w16[i])
                        for k in range(nk):
                            buf[r, pl.ds(k * 16, 16)] = (
                                buf[r, pl.ds(k * 16, 16)] * wv)
                    return 0

                lax.fori_loop(0, EB // 16, sgrp, 0)

                pltpu.sync_copy(buf, acc_sh.at[dst_v.at[j]], add=True)

                @pl.when(j + 2 < NB)
                def _(buf=buf, sem=sem, j=j):
                    pltpu.async_copy(h_hbm.at[src_v.at[j + 2]], buf, sem)

        plsc.subcore_barrier()
        pltpu.sync_copy(acc_sh.at[pl.ds(sid * RPS, RPS)],
                        out_hbm.at[cid, pl.ds(sid * RPS, RPS)])

    return agg_kernel


_agg_f1 = _make_agg(F1)
_agg_f2 = _make_agg(F2P)


def _layer1(x_pad, W0, deg0, deg1):
    def body(x_ref, w_ref, d0_ref, d1_ref, dis_ref, h_ref):
        deg = d0_ref[...] + d1_ref[...] + 1.0
        dis = lax.rsqrt(deg)
        dis_ref[...] = dis
        h = jnp.dot(x_ref[...], w_ref[...], preferred_element_type=jnp.float32)
        h_ref[...] = dis * h

    return pl.pallas_call(
        body,
        grid=(GRID,),
        in_specs=[
            pl.BlockSpec((ROWB, D_IN), lambda i: (i, 0)),
            pl.BlockSpec((D_IN, F1), lambda i: (0, 0)),
            pl.BlockSpec((ROWB, 1), lambda i: (i, 0)),
            pl.BlockSpec((ROWB, 1), lambda i: (i, 0)),
        ],
        out_specs=[
            pl.BlockSpec((ROWB, 1), lambda i: (i, 0)),
            pl.BlockSpec((ROWB, F1), lambda i: (i, 0)),
        ],
        out_shape=[
            jax.ShapeDtypeStruct((NPAD, 1), jnp.float32),
            jax.ShapeDtypeStruct((NPAD, F1), jnp.float32),
        ],
    )(x_pad, W0, deg0, deg1)


def _layer2(a0, a1, h1p, dis, W1p):
    def body(a0_ref, a1_ref, h_ref, dis_ref, w_ref, out_ref):
        dis = dis_ref[...]
        t = jnp.maximum(dis * (a0_ref[...] + a1_ref[...] + h_ref[...]), 0.0)
        out_ref[...] = dis * jnp.dot(t, w_ref[...],
                                     preferred_element_type=jnp.float32)

    return pl.pallas_call(
        body,
        grid=(GRID,),
        in_specs=[
            pl.BlockSpec((ROWB, F1), lambda i: (i, 0)),
            pl.BlockSpec((ROWB, F1), lambda i: (i, 0)),
            pl.BlockSpec((ROWB, F1), lambda i: (i, 0)),
            pl.BlockSpec((ROWB, 1), lambda i: (i, 0)),
            pl.BlockSpec((F1, F2P), lambda i: (0, 0)),
        ],
        out_specs=pl.BlockSpec((ROWB, F2P), lambda i: (i, 0)),
        out_shape=jax.ShapeDtypeStruct((NPAD, F2P), jnp.float32),
    )(a0, a1, h1p, dis, W1p)


def _final(a0, a1, h2p, dis):
    def body(a0_ref, a1_ref, h_ref, dis_ref, out_ref):
        dis = dis_ref[...]
        out_ref[...] = jnp.maximum(dis * (a0_ref[...] + a1_ref[...] + h_ref[...]),
                                   0.0)

    return pl.pallas_call(
        body,
        grid=(GRID,),
        in_specs=[
            pl.BlockSpec((ROWB, F2P), lambda i: (i, 0)),
            pl.BlockSpec((ROWB, F2P), lambda i: (i, 0)),
            pl.BlockSpec((ROWB, F2P), lambda i: (i, 0)),
            pl.BlockSpec((ROWB, 1), lambda i: (i, 0)),
        ],
        out_specs=pl.BlockSpec((ROWB, F2P), lambda i: (i, 0)),
        out_shape=jax.ShapeDtypeStruct((NPAD, F2P), jnp.float32),
    )(a0, a1, h2p, dis)


def kernel(x, edge_index, edge_weight, W0, W1):
    src = edge_index[0].astype(jnp.int32)
    dst = edge_index[1].astype(jnp.int32)
    w = edge_weight.astype(jnp.float32)

    pad = EPAD - src.shape[0]
    src_p = jnp.pad(src, (0, pad)).reshape(NW, NB, EB)
    dst_p = jnp.pad(dst, (0, pad)).reshape(NW, NB, EB)
    w_p = jnp.pad(w, (0, pad)).reshape(NW, NB, EB)
    x_pad = jnp.pad(x, ((0, NPAD - N), (0, 0)))
    W1p = jnp.pad(W1, ((0, 0), (0, F2P - F2)))

    deg_p = _deg_kernel(dst_p, w_p)
    deg0 = deg_p[0].reshape(NPAD, 1)
    deg1 = deg_p[1].reshape(NPAD, 1)

    dis, h1p = _layer1(x_pad, W0, deg0, deg1)
    acc1 = _agg_f1(h1p, src_p, dst_p, w_p)
    h2p = _layer2(acc1[0], acc1[1], h1p, dis, W1p)
    acc2 = _agg_f2(h2p, src_p, dst_p, w_p)
    outp = _final(acc2[0], acc2[1], h2p, dis)
    return outp[:N, :F2]


# speedup vs baseline: 18.2254x; 1.0625x over previous
import functools

import jax
import jax.numpy as jnp
from jax import lax
from jax.experimental import pallas as pl
from jax.experimental.pallas import tpu as pltpu
from jax.experimental.pallas import tpu_sc as plsc

N = 10000
D_IN = 128
F1 = 48
F2 = 60
F2P = 64

NC = 2
NS = 16
NW = NC * NS

NPAD = 10240
RPS = NPAD // NS

EB = 128
NB = 80
EPT = NB * EB
EPAD = NW * EPT

ROWB = 256
GRID = NPAD // ROWB

_MESH = plsc.VectorSubcoreMesh(core_axis_name="c", subcore_axis_name="s")


@functools.partial(
    pl.kernel,
    out_type=jax.ShapeDtypeStruct((NC, NPAD), jnp.float32),
    mesh=_MESH,
    compiler_params=pltpu.CompilerParams(use_tc_tiling_on_sc=False),
    scratch_types=[
        pltpu.VMEM((NB, EB), jnp.int32),
        pltpu.VMEM((NB, EB), jnp.float32),
        pltpu.VMEM((RPS,), jnp.float32),
        pltpu.VMEM_SHARED((NPAD,), jnp.float32),
    ],
)
def _deg_kernel(dst_hbm, w_hbm, out_hbm, dst_v, w_v, zbuf, deg_sh):
    cid = lax.axis_index("c")
    sid = lax.axis_index("s")
    wid = cid * NS + sid
    pltpu.sync_copy(dst_hbm.at[wid], dst_v)
    pltpu.sync_copy(w_hbm.at[wid], w_v)

    zero16 = jnp.zeros((16,), jnp.float32)

    @pl.loop(0, RPS // 16)
    def _(i):
        zbuf[pl.ds(i * 16, 16)] = zero16

    pltpu.sync_copy(zbuf, deg_sh.at[pl.ds(sid * RPS, RPS)])
    plsc.subcore_barrier()

    @pl.loop(0, NB)
    def _(j):
        pltpu.sync_copy(w_v.at[j], deg_sh.at[dst_v.at[j]], add=True)

    plsc.subcore_barrier()
    pltpu.sync_copy(deg_sh.at[pl.ds(sid * RPS, RPS)],
                    out_hbm.at[cid, pl.ds(sid * RPS, RPS)])


def _make_agg(fp):
    nk = fp // 16

    @functools.partial(
        pl.kernel,
        out_type=jax.ShapeDtypeStruct((NC, NPAD, fp), jnp.float32),
        mesh=_MESH,
        compiler_params=pltpu.CompilerParams(use_tc_tiling_on_sc=False),
        scratch_types=[
            pltpu.VMEM((NB, EB), jnp.int32),
            pltpu.VMEM((NB, EB), jnp.int32),
            pltpu.VMEM((NB, EB), jnp.float32),
            pltpu.VMEM((EB, fp), jnp.float32),
            pltpu.VMEM((EB, fp), jnp.float32),
            pltpu.VMEM((EB, fp), jnp.float32),
            pltpu.VMEM((EB, fp), jnp.float32),
            pltpu.VMEM_SHARED((NPAD, fp), jnp.float32),
            pltpu.SemaphoreType.DMA,
            pltpu.SemaphoreType.DMA,
            pltpu.SemaphoreType.DMA,
            pltpu.SemaphoreType.DMA,
        ],
    )
    def agg_kernel(h_hbm, src_hbm, dst_hbm, w_hbm, out_hbm,
                   src_v, dst_v, w_v, g0, g1, s0, s1, acc_sh,
                   gsem0, gsem1, ssem0, ssem1):
        cid = lax.axis_index("c")
        sid = lax.axis_index("s")
        wid = cid * NS + sid
        pltpu.sync_copy(src_hbm.at[wid], src_v)
        pltpu.sync_copy(dst_hbm.at[wid], dst_v)
        pltpu.sync_copy(w_hbm.at[wid], w_v)

        zero16 = jnp.zeros((16,), jnp.float32)

        @pl.loop(0, EB)
        def _(r):
            for k in range(nk):
                g0[r, pl.ds(k * 16, 16)] = zero16

        @pl.loop(0, RPS // EB)
        def _(i):
            pltpu.sync_copy(g0, acc_sh.at[pl.ds(sid * RPS + i * EB, EB)])

        plsc.subcore_barrier()

        gbufs = (g0, g1)
        sbufs = (s0, s1)
        gsems = (gsem0, gsem1)
        ssems = (ssem0, ssem1)
        K = 2
        for b in range(K):
            pltpu.async_copy(h_hbm.at[src_v.at[b]], gbufs[b], gsems[b])

        @pl.loop(0, NB, step=K)
        def _(j0):
            for b in range(K):
                j = j0 + b
                g, s = gbufs[b], sbufs[b]
                pltpu.make_async_copy(h_hbm.at[src_v.at[j]], g, gsems[b]).wait()

                @pl.when(j >= K)
                def _(s=s, b=b, j=j):
                    pltpu.make_async_copy(
                        s, acc_sh.at[dst_v.at[j - K]], ssems[b]).wait()

                @plsc.parallel_loop(0, EB // 16, unroll=2)
                def _(gi, g=g, s=s, j=j):
                    w16 = w_v[j, pl.ds(gi * 16, 16)]
                    for i in range(16):
                        r = gi * 16 + i
                        wv = jnp.full((16,), w16[i])
                        for k in range(nk):
                            s[r, pl.ds(k * 16, 16)] = (
                                g[r, pl.ds(k * 16, 16)] * wv)

                @pl.when(j + K < NB)
                def _(g=g, b=b, j=j):
                    pltpu.async_copy(h_hbm.at[src_v.at[j + K]], g, gsems[b])

                pltpu.async_copy(s, acc_sh.at[dst_v.at[j]], ssems[b], add=True)

        for b in range(K):
            pltpu.make_async_copy(
                sbufs[b], acc_sh.at[dst_v.at[NB - K + b]], ssems[b]).wait()

        plsc.subcore_barrier()
        pltpu.sync_copy(acc_sh.at[pl.ds(sid * RPS, RPS)],
                        out_hbm.at[cid, pl.ds(sid * RPS, RPS)])

    return agg_kernel


_agg_f1 = _make_agg(F1)
_agg_f2 = _make_agg(F2P)


def _layer1(x_pad, W0, deg0, deg1):
    def body(x_ref, w_ref, d0_ref, d1_ref, dis_ref, h_ref):
        deg = d0_ref[...] + d1_ref[...] + 1.0
        dis = lax.rsqrt(deg)
        dis_ref[...] = dis
        h = jnp.dot(x_ref[...], w_ref[...], preferred_element_type=jnp.float32)
        h_ref[...] = dis * h

    return pl.pallas_call(
        body,
        grid=(GRID,),
        in_specs=[
            pl.BlockSpec((ROWB, D_IN), lambda i: (i, 0)),
            pl.BlockSpec((D_IN, F1), lambda i: (0, 0)),
            pl.BlockSpec((ROWB, 1), lambda i: (i, 0)),
            pl.BlockSpec((ROWB, 1), lambda i: (i, 0)),
        ],
        out_specs=[
            pl.BlockSpec((ROWB, 1), lambda i: (i, 0)),
            pl.BlockSpec((ROWB, F1), lambda i: (i, 0)),
        ],
        out_shape=[
            jax.ShapeDtypeStruct((NPAD, 1), jnp.float32),
            jax.ShapeDtypeStruct((NPAD, F1), jnp.float32),
        ],
    )(x_pad, W0, deg0, deg1)


def _layer2(a0, a1, h1p, dis, W1p):
    def body(a0_ref, a1_ref, h_ref, dis_ref, w_ref, out_ref):
        dis = dis_ref[...]
        t = jnp.maximum(dis * (a0_ref[...] + a1_ref[...] + h_ref[...]), 0.0)
        out_ref[...] = dis * jnp.dot(t, w_ref[...],
                                     preferred_element_type=jnp.float32)

    return pl.pallas_call(
        body,
        grid=(GRID,),
        in_specs=[
            pl.BlockSpec((ROWB, F1), lambda i: (i, 0)),
            pl.BlockSpec((ROWB, F1), lambda i: (i, 0)),
            pl.BlockSpec((ROWB, F1), lambda i: (i, 0)),
            pl.BlockSpec((ROWB, 1), lambda i: (i, 0)),
            pl.BlockSpec((F1, F2P), lambda i: (0, 0)),
        ],
        out_specs=pl.BlockSpec((ROWB, F2P), lambda i: (i, 0)),
        out_shape=jax.ShapeDtypeStruct((NPAD, F2P), jnp.float32),
    )(a0, a1, h1p, dis, W1p)


def _final(a0, a1, h2p, dis):
    def body(a0_ref, a1_ref, h_ref, dis_ref, out_ref):
        dis = dis_ref[...]
        out_ref[...] = jnp.maximum(dis * (a0_ref[...] + a1_ref[...] + h_ref[...]),
                                   0.0)

    return pl.pallas_call(
        body,
        grid=(GRID,),
        in_specs=[
            pl.BlockSpec((ROWB, F2P), lambda i: (i, 0)),
            pl.BlockSpec((ROWB, F2P), lambda i: (i, 0)),
            pl.BlockSpec((ROWB, F2P), lambda i: (i, 0)),
            pl.BlockSpec((ROWB, 1), lambda i: (i, 0)),
        ],
        out_specs=pl.BlockSpec((ROWB, F2P), lambda i: (i, 0)),
        out_shape=jax.ShapeDtypeStruct((NPAD, F2P), jnp.float32),
    )(a0, a1, h2p, dis)


def kernel(x, edge_index, edge_weight, W0, W1):
    src = edge_index[0].astype(jnp.int32)
    dst = edge_index[1].astype(jnp.int32)
    w = edge_weight.astype(jnp.float32)

    pad = EPAD - src.shape[0]
    src_p = jnp.pad(src, (0, pad)).reshape(NW, NB, EB)
    dst_p = jnp.pad(dst, (0, pad)).reshape(NW, NB, EB)
    w_p = jnp.pad(w, (0, pad)).reshape(NW, NB, EB)
    x_pad = jnp.pad(x, ((0, NPAD - N), (0, 0)))
    W1p = jnp.pad(W1, ((0, 0), (0, F2P - F2)))

    deg_p = _deg_kernel(dst_p, w_p)
    deg0 = deg_p[0].reshape(NPAD, 1)
    deg1 = deg_p[1].reshape(NPAD, 1)

    dis, h1p = _layer1(x_pad, W0, deg0, deg1)
    acc1 = _agg_f1(h1p, src_p, dst_p, w_p)
    h2p = _layer2(acc1[0], acc1[1], h1p, dis, W1p)
    acc2 = _agg_f2(h2p, src_p, dst_p, w_p)
    outp = _final(acc2[0], acc2[1], h2p, dis)
    return outp[:N, :F2]


# speedup vs baseline: 19.4143x; 1.0652x over previous
import functools

import jax
import jax.numpy as jnp
from jax import lax
from jax.experimental import pallas as pl
from jax.experimental.pallas import tpu as pltpu
from jax.experimental.pallas import tpu_sc as plsc

N = 10000
D_IN = 128
F1 = 48
F2 = 60
F2P = 64

NC = 2
NS = 16
NW = NC * NS

NPAD = 10240
RPS = NPAD // NS

EB = 128
NB = 80
EPT = NB * EB
EPAD = NW * EPT

ROWB = 2000
GRID = N // ROWB

_MESH = plsc.VectorSubcoreMesh(core_axis_name="c", subcore_axis_name="s")


@functools.partial(
    pl.kernel,
    out_type=jax.ShapeDtypeStruct((NC, NPAD), jnp.float32),
    mesh=_MESH,
    compiler_params=pltpu.CompilerParams(use_tc_tiling_on_sc=False),
    scratch_types=[
        pltpu.VMEM((NB, EB), jnp.int32),
        pltpu.VMEM((NB, EB), jnp.float32),
        pltpu.VMEM((RPS,), jnp.float32),
        pltpu.VMEM_SHARED((NPAD,), jnp.float32),
    ],
)
def _deg_kernel(dst_hbm, w_hbm, out_hbm, dst_v, w_v, zbuf, deg_sh):
    cid = lax.axis_index("c")
    sid = lax.axis_index("s")
    wid = cid * NS + sid
    pltpu.sync_copy(dst_hbm.at[wid], dst_v)
    pltpu.sync_copy(w_hbm.at[wid], w_v)

    zero16 = jnp.zeros((16,), jnp.float32)

    @pl.loop(0, RPS // 16)
    def _(i):
        zbuf[pl.ds(i * 16, 16)] = zero16

    pltpu.sync_copy(zbuf, deg_sh.at[pl.ds(sid * RPS, RPS)])
    plsc.subcore_barrier()

    @pl.loop(0, NB)
    def _(j):
        pltpu.sync_copy(w_v.at[j], deg_sh.at[dst_v.at[j]], add=True)

    plsc.subcore_barrier()
    pltpu.sync_copy(deg_sh.at[pl.ds(sid * RPS, RPS)],
                    out_hbm.at[cid, pl.ds(sid * RPS, RPS)])


def _make_agg(fp):
    nk = fp // 16

    @functools.partial(
        pl.kernel,
        out_type=jax.ShapeDtypeStruct((NC, NPAD, fp), jnp.float32),
        mesh=_MESH,
        compiler_params=pltpu.CompilerParams(use_tc_tiling_on_sc=False),
        scratch_types=[
            pltpu.VMEM((NB, EB), jnp.int32),
            pltpu.VMEM((NB, EB), jnp.int32),
            pltpu.VMEM((NB, EB), jnp.float32),
            pltpu.VMEM((EB, fp), jnp.float32),
            pltpu.VMEM((EB, fp), jnp.float32),
            pltpu.VMEM((EB, fp), jnp.float32),
            pltpu.VMEM((EB, fp), jnp.float32),
            pltpu.VMEM_SHARED((NPAD, fp), jnp.float32),
            pltpu.SemaphoreType.DMA,
            pltpu.SemaphoreType.DMA,
            pltpu.SemaphoreType.DMA,
            pltpu.SemaphoreType.DMA,
        ],
    )
    def agg_kernel(h_hbm, src_hbm, dst_hbm, w_hbm, out_hbm,
                   src_v, dst_v, w_v, g0, g1, s0, s1, acc_sh,
                   gsem0, gsem1, ssem0, ssem1):
        cid = lax.axis_index("c")
        sid = lax.axis_index("s")
        wid = cid * NS + sid
        pltpu.sync_copy(src_hbm.at[wid], src_v)
        pltpu.sync_copy(dst_hbm.at[wid], dst_v)
        pltpu.sync_copy(w_hbm.at[wid], w_v)

        zero16 = jnp.zeros((16,), jnp.float32)

        @pl.loop(0, EB)
        def _(r):
            for k in range(nk):
                g0[r, pl.ds(k * 16, 16)] = zero16

        @pl.loop(0, RPS // EB)
        def _(i):
            pltpu.sync_copy(g0, acc_sh.at[pl.ds(sid * RPS + i * EB, EB)])

        plsc.subcore_barrier()

        gbufs = (g0, g1)
        sbufs = (s0, s1)
        gsems = (gsem0, gsem1)
        ssems = (ssem0, ssem1)
        K = 2
        for b in range(K):
            pltpu.async_copy(h_hbm.at[src_v.at[b]], gbufs[b], gsems[b])

        @pl.loop(0, NB, step=K)
        def _(j0):
            for b in range(K):
                j = j0 + b
                g, s = gbufs[b], sbufs[b]
                pltpu.make_async_copy(h_hbm.at[src_v.at[j]], g, gsems[b]).wait()

                @pl.when(j >= K)
                def _(s=s, b=b, j=j):
                    pltpu.make_async_copy(
                        s, acc_sh.at[dst_v.at[j - K]], ssems[b]).wait()

                @plsc.parallel_loop(0, EB // 16, unroll=2)
                def _(gi, g=g, s=s, j=j):
                    w16 = w_v[j, pl.ds(gi * 16, 16)]
                    for i in range(16):
                        r = gi * 16 + i
                        wv = jnp.full((16,), w16[i])
                        for k in range(nk):
                            s[r, pl.ds(k * 16, 16)] = (
                                g[r, pl.ds(k * 16, 16)] * wv)

                @pl.when(j + K < NB)
                def _(g=g, b=b, j=j):
                    pltpu.async_copy(h_hbm.at[src_v.at[j + K]], g, gsems[b])

                pltpu.async_copy(s, acc_sh.at[dst_v.at[j]], ssems[b], add=True)

        for b in range(K):
            pltpu.make_async_copy(
                sbufs[b], acc_sh.at[dst_v.at[NB - K + b]], ssems[b]).wait()

        plsc.subcore_barrier()
        pltpu.sync_copy(acc_sh.at[pl.ds(sid * RPS, RPS)],
                        out_hbm.at[cid, pl.ds(sid * RPS, RPS)])

    return agg_kernel


_agg_f1 = _make_agg(F1)
_agg_f2 = _make_agg(F2P)


def _layer1(x, W0, deg_p):
    def body(x_ref, w_ref, d_ref, dis_ref, h_ref):
        deg = d_ref[0, 0, :, :] + d_ref[0, 1, :, :] + 1.0
        dis = lax.rsqrt(deg)
        dis_ref[...] = dis
        h = jnp.dot(x_ref[...], w_ref[...], preferred_element_type=jnp.float32)
        h_ref[...] = dis * h

    return pl.pallas_call(
        body,
        grid=(GRID,),
        in_specs=[
            pl.BlockSpec((ROWB, D_IN), lambda i: (i, 0)),
            pl.BlockSpec((D_IN, F1), lambda i: (0, 0)),
            pl.BlockSpec((1, NC, ROWB, 1), lambda i: (0, 0, i, 0)),
        ],
        out_specs=[
            pl.BlockSpec((ROWB, 1), lambda i: (i, 0)),
            pl.BlockSpec((ROWB, F1), lambda i: (i, 0)),
        ],
        out_shape=[
            jax.ShapeDtypeStruct((N, 1), jnp.float32),
            jax.ShapeDtypeStruct((N, F1), jnp.float32),
        ],
    )(x, W0, deg_p)


def _layer2(acc, h1p, dis, W1p):
    def body(a0_ref, a1_ref, h_ref, dis_ref, w_ref, out_ref):
        dis = dis_ref[...]
        t = jnp.maximum(
            dis * (a0_ref[0, :, :] + a1_ref[0, :, :] + h_ref[...]), 0.0)
        out_ref[...] = dis * jnp.dot(t, w_ref[...],
                                     preferred_element_type=jnp.float32)

    return pl.pallas_call(
        body,
        grid=(GRID,),
        in_specs=[
            pl.BlockSpec((1, ROWB, F1), lambda i: (0, i, 0)),
            pl.BlockSpec((1, ROWB, F1), lambda i: (1, i, 0)),
            pl.BlockSpec((ROWB, F1), lambda i: (i, 0)),
            pl.BlockSpec((ROWB, 1), lambda i: (i, 0)),
            pl.BlockSpec((F1, F2P), lambda i: (0, 0)),
        ],
        out_specs=pl.BlockSpec((ROWB, F2P), lambda i: (i, 0)),
        out_shape=jax.ShapeDtypeStruct((N, F2P), jnp.float32),
    )(acc, acc, h1p, dis, W1p)


def _final(acc, h2p, dis):
    def body(a0_ref, a1_ref, h_ref, dis_ref, out_ref):
        dis = dis_ref[...]
        v = jnp.maximum(
            dis * (a0_ref[0, :, :] + a1_ref[0, :, :] + h_ref[...]), 0.0)
        out_ref[...] = v[:, :F2]

    return pl.pallas_call(
        body,
        grid=(GRID,),
        in_specs=[
            pl.BlockSpec((1, ROWB, F2P), lambda i: (0, i, 0)),
            pl.BlockSpec((1, ROWB, F2P), lambda i: (1, i, 0)),
            pl.BlockSpec((ROWB, F2P), lambda i: (i, 0)),
            pl.BlockSpec((ROWB, 1), lambda i: (i, 0)),
        ],
        out_specs=pl.BlockSpec((ROWB, F2), lambda i: (i, 0)),
        out_shape=jax.ShapeDtypeStruct((N, F2), jnp.float32),
    )(acc, acc, h2p, dis)


def kernel(x, edge_index, edge_weight, W0, W1):
    src = edge_index[0].astype(jnp.int32)
    dst = edge_index[1].astype(jnp.int32)
    w = edge_weight.astype(jnp.float32)

    pad = EPAD - src.shape[0]
    src_p = jnp.pad(src, (0, pad)).reshape(NW, NB, EB)
    dst_p = jnp.pad(dst, (0, pad)).reshape(NW, NB, EB)
    w_p = jnp.pad(w, (0, pad)).reshape(NW, NB, EB)
    W1p = jnp.pad(W1, ((0, 0), (0, F2P - F2)))

    deg_p = _deg_kernel(dst_p, w_p).reshape(1, NC, NPAD, 1)

    dis, h1p = _layer1(x, W0, deg_p)
    acc1 = _agg_f1(h1p, src_p, dst_p, w_p)
    h2p = _layer2(acc1, h1p, dis, W1p)
    acc2 = _agg_f2(h2p, src_p, dst_p, w_p)
    return _final(acc2, h2p, dis)


# speedup vs baseline: 33.8089x; 1.7414x over previous
import functools

import jax
import jax.numpy as jnp
from jax import lax
from jax.experimental import pallas as pl
from jax.experimental.pallas import tpu as pltpu
from jax.experimental.pallas import tpu_sc as plsc

N = 10000
D_IN = 128
F1 = 48
F1H = F1 // 2
F2 = 60
F2P = 64
F2H = F2P // 2

NC = 2
NS = 16

NPAD = 10240
RPS = NPAD // NS
RH = N // NS

EB = 128
NB = 160
EPAD = NS * NB * EB

ROWB = 2000
GRID = N // ROWB

_MESH = plsc.VectorSubcoreMesh(core_axis_name="c", subcore_axis_name="s")


def _vslices(fph):
    starts = list(range(0, fph - 15, 16))
    if fph % 16:
        starts.append(fph - 16)
    return starts


@functools.partial(
    pl.kernel,
    out_type=jax.ShapeDtypeStruct((NC, NPAD), jnp.float32),
    mesh=_MESH,
    compiler_params=pltpu.CompilerParams(use_tc_tiling_on_sc=False),
    scratch_types=[
        pltpu.VMEM((NB // 2, EB), jnp.int32),
        pltpu.VMEM((NB // 2, EB), jnp.float32),
        pltpu.VMEM((RPS,), jnp.float32),
        pltpu.VMEM_SHARED((NPAD,), jnp.float32),
    ],
)
def _deg_kernel(dst_hbm, w_hbm, out_hbm, dst_v, w_v, zbuf, deg_sh):
    cid = lax.axis_index("c")
    sid = lax.axis_index("s")
    half = NB // 2
    pltpu.sync_copy(dst_hbm.at[sid, pl.ds(cid * half, half)], dst_v)
    pltpu.sync_copy(w_hbm.at[sid, pl.ds(cid * half, half)], w_v)

    zero16 = jnp.zeros((16,), jnp.float32)

    @pl.loop(0, RPS // 16)
    def _(i):
        zbuf[pl.ds(i * 16, 16)] = zero16

    pltpu.sync_copy(zbuf, deg_sh.at[pl.ds(sid * RPS, RPS)])
    plsc.subcore_barrier()

    @pl.loop(0, half)
    def _(j):
        pltpu.sync_copy(w_v.at[j], deg_sh.at[dst_v.at[j]], add=True)

    plsc.subcore_barrier()
    pltpu.sync_copy(deg_sh.at[pl.ds(sid * RPS, RPS)],
                    out_hbm.at[cid, pl.ds(sid * RPS, RPS)])


def _make_agg(fph):
    starts = _vslices(fph)

    @functools.partial(
        pl.kernel,
        out_type=jax.ShapeDtypeStruct((NC, NPAD, fph), jnp.float32),
        mesh=_MESH,
        compiler_params=pltpu.CompilerParams(use_tc_tiling_on_sc=False),
        scratch_types=[
            pltpu.VMEM((NB, EB), jnp.int32),
            pltpu.VMEM((NB, EB), jnp.int32),
            pltpu.VMEM((NB, EB), jnp.float32),
            pltpu.VMEM((EB, fph), jnp.float32),
            pltpu.VMEM((EB, fph), jnp.float32),
            pltpu.VMEM((EB, fph), jnp.float32),
            pltpu.VMEM((EB, fph), jnp.float32),
            pltpu.VMEM_SHARED((NPAD, fph), jnp.float32),
            pltpu.VMEM_SHARED((N, fph), jnp.float32),
            pltpu.SemaphoreType.DMA,
            pltpu.SemaphoreType.DMA,
            pltpu.SemaphoreType.DMA,
            pltpu.SemaphoreType.DMA,
        ],
    )
    def agg_kernel(h_hbm, src_hbm, dst_hbm, w_hbm, out_hbm,
                   src_v, dst_v, w_v, g0, g1, s0, s1, acc_sh, h_sh,
                   gsem0, gsem1, ssem0, ssem1):
        cid = lax.axis_index("c")
        sid = lax.axis_index("s")
        pltpu.sync_copy(src_hbm.at[sid], src_v)
        pltpu.sync_copy(dst_hbm.at[sid], dst_v)
        pltpu.sync_copy(w_hbm.at[sid], w_v)

        pltpu.sync_copy(h_hbm.at[cid, pl.ds(sid * RH, RH)],
                        h_sh.at[pl.ds(sid * RH, RH)])

        zero16 = jnp.zeros((16,), jnp.float32)

        @pl.loop(0, EB)
        def _(r):
            for st in starts:
                g0[r, pl.ds(st, 16)] = zero16

        @pl.loop(0, RPS // EB)
        def _(i):
            pltpu.sync_copy(g0, acc_sh.at[pl.ds(sid * RPS + i * EB, EB)])

        plsc.subcore_barrier()

        gbufs = (g0, g1)
        sbufs = (s0, s1)
        gsems = (gsem0, gsem1)
        ssems = (ssem0, ssem1)
        K = 2
        for b in range(K):
            pltpu.async_copy(h_sh.at[src_v.at[b]], gbufs[b], gsems[b])

        @pl.loop(0, NB, step=K)
        def _(j0):
            for b in range(K):
                j = j0 + b
                g, s = gbufs[b], sbufs[b]
                pltpu.make_async_copy(h_sh.at[src_v.at[j]], g, gsems[b]).wait()

                @pl.when(j >= K)
                def _(s=s, b=b, j=j):
                    pltpu.make_async_copy(
                        s, acc_sh.at[dst_v.at[j - K]], ssems[b]).wait()

                @plsc.parallel_loop(0, EB // 16, unroll=2)
                def _(gi, g=g, s=s, j=j):
                    w16 = w_v[j, pl.ds(gi * 16, 16)]
                    for i in range(16):
                        r = gi * 16 + i
                        wv = jnp.full((16,), w16[i])
                        for st in starts:
                            s[r, pl.ds(st, 16)] = g[r, pl.ds(st, 16)] * wv

                @pl.when(j + K < NB)
                def _(g=g, b=b, j=j):
                    pltpu.async_copy(h_sh.at[src_v.at[j + K]], g, gsems[b])

                pltpu.async_copy(s, acc_sh.at[dst_v.at[j]], ssems[b], add=True)

        for b in range(K):
            pltpu.make_async_copy(
                sbufs[b], acc_sh.at[dst_v.at[NB - K + b]], ssems[b]).wait()

        plsc.subcore_barrier()
        pltpu.sync_copy(acc_sh.at[pl.ds(sid * RPS, RPS)],
                        out_hbm.at[cid, pl.ds(sid * RPS, RPS)])

    return agg_kernel


_agg_l1 = _make_agg(F1H)
_agg_l2 = _make_agg(F2H)


def _layer1(x, W0s, deg_p):
    def body(x_ref, w_ref, d_ref, dis_ref, h_ref):
        deg = d_ref[0, 0, :, :] + d_ref[0, 1, :, :] + 1.0
        dis = lax.rsqrt(deg)
        dis_ref[...] = dis
        h = jnp.dot(x_ref[...], w_ref[0], preferred_element_type=jnp.float32)
        h_ref[0, :, :] = dis * h

    return pl.pallas_call(
        body,
        grid=(GRID, NC),
        in_specs=[
            pl.BlockSpec((ROWB, D_IN), lambda i, c: (i, 0)),
            pl.BlockSpec((1, D_IN, F1H), lambda i, c: (c, 0, 0)),
            pl.BlockSpec((1, NC, ROWB, 1), lambda i, c: (0, 0, i, 0)),
        ],
        out_specs=[
            pl.BlockSpec((ROWB, 1), lambda i, c: (i, 0)),
            pl.BlockSpec((1, ROWB, F1H), lambda i, c: (c, i, 0)),
        ],
        out_shape=[
            jax.ShapeDtypeStruct((N, 1), jnp.float32),
            jax.ShapeDtypeStruct((NC, N, F1H), jnp.float32),
        ],
    )(x, W0s, deg_p)


def _layer2(acc, h1p, dis, W1s):
    def body(a0_ref, a1_ref, h0_ref, h1_ref, dis_ref, w0_ref, w1_ref, out_ref):
        dis = dis_ref[...]
        t0 = jnp.maximum(dis * (a0_ref[0, :, :] + h0_ref[0, :, :]), 0.0)
        t1 = jnp.maximum(dis * (a1_ref[0, :, :] + h1_ref[0, :, :]), 0.0)
        v = (jnp.dot(t0, w0_ref[0, 0], preferred_element_type=jnp.float32)
             + jnp.dot(t1, w1_ref[0, 0], preferred_element_type=jnp.float32))
        out_ref[0, :, :] = dis * v

    return pl.pallas_call(
        body,
        grid=(GRID, NC),
        in_specs=[
            pl.BlockSpec((1, ROWB, F1H), lambda i, c: (0, i, 0)),
            pl.BlockSpec((1, ROWB, F1H), lambda i, c: (1, i, 0)),
            pl.BlockSpec((1, ROWB, F1H), lambda i, c: (0, i, 0)),
            pl.BlockSpec((1, ROWB, F1H), lambda i, c: (1, i, 0)),
            pl.BlockSpec((ROWB, 1), lambda i, c: (i, 0)),
            pl.BlockSpec((1, 1, F1H, F2H), lambda i, c: (c, 0, 0, 0)),
            pl.BlockSpec((1, 1, F1H, F2H), lambda i, c: (c, 1, 0, 0)),
        ],
        out_specs=pl.BlockSpec((1, ROWB, F2H), lambda i, c: (c, i, 0)),
        out_shape=jax.ShapeDtypeStruct((NC, N, F2H), jnp.float32),
    )(acc, acc, h1p, h1p, dis, W1s, W1s)


def _final(acc, h2p, dis):
    def body(a0_ref, a1_ref, h0_ref, h1_ref, dis_ref, out_ref):
        dis = dis_ref[...]
        v0 = jnp.maximum(dis * (a0_ref[0, :, :] + h0_ref[0, :, :]), 0.0)
        v1 = jnp.maximum(dis * (a1_ref[0, :, :] + h1_ref[0, :, :]), 0.0)
        out_ref[:, :F2H] = v0
        out_ref[:, F2H:] = v1[:, :F2 - F2H]

    return pl.pallas_call(
        body,
        grid=(GRID,),
        in_specs=[
            pl.BlockSpec((1, ROWB, F2H), lambda i: (0, i, 0)),
            pl.BlockSpec((1, ROWB, F2H), lambda i: (1, i, 0)),
            pl.BlockSpec((1, ROWB, F2H), lambda i: (0, i, 0)),
            pl.BlockSpec((1, ROWB, F2H), lambda i: (1, i, 0)),
            pl.BlockSpec((ROWB, 1), lambda i: (i, 0)),
        ],
        out_specs=pl.BlockSpec((ROWB, F2), lambda i: (i, 0)),
        out_shape=jax.ShapeDtypeStruct((N, F2), jnp.float32),
    )(acc, acc, h2p, h2p, dis)


def kernel(x, edge_index, edge_weight, W0, W1):
    src = edge_index[0].astype(jnp.int32)
    dst = edge_index[1].astype(jnp.int32)
    w = edge_weight.astype(jnp.float32)

    pad = EPAD - src.shape[0]
    src_p = jnp.pad(src, (0, pad)).reshape(NS, NB, EB)
    dst_p = jnp.pad(dst, (0, pad)).reshape(NS, NB, EB)
    w_p = jnp.pad(w, (0, pad)).reshape(NS, NB, EB)
    W1s = (jnp.pad(W1, ((0, 0), (0, F2P - F2)))
           .reshape(NC, F1H, NC, F2H).transpose(2, 0, 1, 3))

    deg_p = _deg_kernel(dst_p, w_p).reshape(1, NC, NPAD, 1)

    W0s = W0.reshape(D_IN, NC, F1H).transpose(1, 0, 2)
    dis, h1p = _layer1(x, W0s, deg_p)
    acc1 = _agg_l1(h1p, src_p, dst_p, w_p)
    h2p = _layer2(acc1, h1p, dis, W1s)
    acc2 = _agg_l2(h2p, src_p, dst_p, w_p)
    return _final(acc2, h2p, dis)


# speedup vs baseline: 34.2522x; 1.0131x over previous
import functools

import jax
import jax.numpy as jnp
from jax import lax
from jax.experimental import pallas as pl
from jax.experimental.pallas import tpu as pltpu
from jax.experimental.pallas import tpu_sc as plsc

N = 10000
D_IN = 128
F1 = 48
F1H = F1 // 2
F2 = 60
F2P = 64
F2H = F2P // 2

NC = 2
NS = 16

NPAD = 10240
RPS = NPAD // NS
RH = N // NS

EB = 128
NB = 160
EPAD = NS * NB * EB

ROWB = 2000
GRID = N // ROWB

_MESH = plsc.VectorSubcoreMesh(core_axis_name="c", subcore_axis_name="s")


def _vslices(fph):
    starts = list(range(0, fph - 15, 16))
    if fph % 16:
        starts.append(fph - 16)
    return starts


@functools.partial(
    pl.kernel,
    out_type=jax.ShapeDtypeStruct((NC, NPAD), jnp.float32),
    mesh=_MESH,
    compiler_params=pltpu.CompilerParams(use_tc_tiling_on_sc=False),
    scratch_types=[
        pltpu.VMEM((NB // 2, EB), jnp.int32),
        pltpu.VMEM((NB // 2, EB), jnp.float32),
        pltpu.VMEM((RPS,), jnp.float32),
        pltpu.VMEM_SHARED((NPAD,), jnp.float32),
    ],
)
def _deg_kernel(dst_hbm, w_hbm, out_hbm, dst_v, w_v, zbuf, deg_sh):
    cid = lax.axis_index("c")
    sid = lax.axis_index("s")
    half = NB // 2
    pltpu.sync_copy(dst_hbm.at[sid, pl.ds(cid * half, half)], dst_v)
    pltpu.sync_copy(w_hbm.at[sid, pl.ds(cid * half, half)], w_v)

    zero16 = jnp.zeros((16,), jnp.float32)

    @pl.loop(0, RPS // 16)
    def _(i):
        zbuf[pl.ds(i * 16, 16)] = zero16

    pltpu.sync_copy(zbuf, deg_sh.at[pl.ds(sid * RPS, RPS)])
    plsc.subcore_barrier()

    @pl.loop(0, half)
    def _(j):
        pltpu.sync_copy(w_v.at[j], deg_sh.at[dst_v.at[j]], add=True)

    plsc.subcore_barrier()
    pltpu.sync_copy(deg_sh.at[pl.ds(sid * RPS, RPS)],
                    out_hbm.at[cid, pl.ds(sid * RPS, RPS)])


def _make_agg(fph):
    starts = _vslices(fph)

    @functools.partial(
        pl.kernel,
        out_type=jax.ShapeDtypeStruct((NC, NPAD, fph), jnp.float32),
        mesh=_MESH,
        compiler_params=pltpu.CompilerParams(use_tc_tiling_on_sc=False),
        scratch_types=[
            pltpu.VMEM((NB, EB), jnp.int32),
            pltpu.VMEM((NB, EB), jnp.int32),
            pltpu.VMEM((NB, EB), jnp.float32),
            pltpu.VMEM((EB, fph), jnp.float32),
            pltpu.VMEM((EB, fph), jnp.float32),
            pltpu.VMEM((EB, fph), jnp.float32),
            pltpu.VMEM((EB, fph), jnp.float32),
            pltpu.VMEM_SHARED((NPAD, fph), jnp.float32),
            pltpu.VMEM_SHARED((N, fph), jnp.float32),
            pltpu.SemaphoreType.DMA,
            pltpu.SemaphoreType.DMA,
            pltpu.SemaphoreType.DMA,
            pltpu.SemaphoreType.DMA,
        ],
    )
    def agg_kernel(h_hbm, src_hbm, dst_hbm, w_hbm, out_hbm,
                   src_v, dst_v, w_v, g0, g1, s0, s1, acc_sh, h_sh,
                   gsem0, gsem1, ssem0, ssem1):
        cid = lax.axis_index("c")
        sid = lax.axis_index("s")
        pltpu.sync_copy(src_hbm.at[sid], src_v)
        pltpu.sync_copy(dst_hbm.at[sid], dst_v)
        pltpu.sync_copy(w_hbm.at[sid], w_v)

        pltpu.sync_copy(h_hbm.at[cid, pl.ds(sid * RH, RH)],
                        h_sh.at[pl.ds(sid * RH, RH)])

        zero16 = jnp.zeros((16,), jnp.float32)

        @pl.loop(0, EB)
        def _(r):
            for st in starts:
                g0[r, pl.ds(st, 16)] = zero16

        @pl.loop(0, RPS // EB)
        def _(i):
            pltpu.sync_copy(g0, acc_sh.at[pl.ds(sid * RPS + i * EB, EB)])

        plsc.subcore_barrier()

        gbufs = (g0, g1)
        sbufs = (s0, s1)
        gsems = (gsem0, gsem1)
        ssems = (ssem0, ssem1)
        K = 2
        for b in range(K):
            pltpu.async_copy(h_sh.at[src_v.at[b]], gbufs[b], gsems[b])

        @pl.loop(0, NB, step=K)
        def _(j0):
            for b in range(K):
                j = j0 + b
                g, s = gbufs[b], sbufs[b]
                pltpu.make_async_copy(h_sh.at[src_v.at[j]], g, gsems[b]).wait()

                @pl.when(j >= K)
                def _(s=s, b=b, j=j):
                    pltpu.make_async_copy(
                        s, acc_sh.at[dst_v.at[j - K]], ssems[b]).wait()

                @plsc.parallel_loop(0, EB // 16, unroll=4)
                def _(gi, g=g, s=s, j=j):
                    w16 = w_v[j, pl.ds(gi * 16, 16)]
                    for i in range(16):
                        r = gi * 16 + i
                        wv = jnp.full((16,), w16[i])
                        for st in starts:
                            s[r, pl.ds(st, 16)] = g[r, pl.ds(st, 16)] * wv

                @pl.when(j + K < NB)
                def _(g=g, b=b, j=j):
                    pltpu.async_copy(h_sh.at[src_v.at[j + K]], g, gsems[b])

                pltpu.async_copy(s, acc_sh.at[dst_v.at[j]], ssems[b], add=True)

        for b in range(K):
            pltpu.make_async_copy(
                sbufs[b], acc_sh.at[dst_v.at[NB - K + b]], ssems[b]).wait()

        plsc.subcore_barrier()
        pltpu.sync_copy(acc_sh.at[pl.ds(sid * RPS, RPS)],
                        out_hbm.at[cid, pl.ds(sid * RPS, RPS)])

    return agg_kernel


_agg_l1 = _make_agg(F1H)
_agg_l2 = _make_agg(F2H)


def _layer1(x, W0s, deg_p):
    def body(x_ref, w_ref, d_ref, dis_ref, h_ref):
        deg = d_ref[0, 0, :, :] + d_ref[0, 1, :, :] + 1.0
        dis = lax.rsqrt(deg)
        dis_ref[...] = dis
        h = jnp.dot(x_ref[...], w_ref[0], preferred_element_type=jnp.float32)
        h_ref[0, :, :] = dis * h

    return pl.pallas_call(
        body,
        grid=(GRID, NC),
        in_specs=[
            pl.BlockSpec((ROWB, D_IN), lambda i, c: (i, 0)),
            pl.BlockSpec((1, D_IN, F1H), lambda i, c: (c, 0, 0)),
            pl.BlockSpec((1, NC, ROWB, 1), lambda i, c: (0, 0, i, 0)),
        ],
        out_specs=[
            pl.BlockSpec((ROWB, 1), lambda i, c: (i, 0)),
            pl.BlockSpec((1, ROWB, F1H), lambda i, c: (c, i, 0)),
        ],
        out_shape=[
            jax.ShapeDtypeStruct((N, 1), jnp.float32),
            jax.ShapeDtypeStruct((NC, N, F1H), jnp.float32),
        ],
    )(x, W0s, deg_p)


def _layer2(acc, h1p, dis, W1s):
    def body(a0_ref, a1_ref, h0_ref, h1_ref, dis_ref, w0_ref, w1_ref, out_ref):
        dis = dis_ref[...]
        t0 = jnp.maximum(dis * (a0_ref[0, :, :] + h0_ref[0, :, :]), 0.0)
        t1 = jnp.maximum(dis * (a1_ref[0, :, :] + h1_ref[0, :, :]), 0.0)
        v = (jnp.dot(t0, w0_ref[0, 0], preferred_element_type=jnp.float32)
             + jnp.dot(t1, w1_ref[0, 0], preferred_element_type=jnp.float32))
        out_ref[0, :, :] = dis * v

    return pl.pallas_call(
        body,
        grid=(GRID, NC),
        in_specs=[
            pl.BlockSpec((1, ROWB, F1H), lambda i, c: (0, i, 0)),
            pl.BlockSpec((1, ROWB, F1H), lambda i, c: (1, i, 0)),
            pl.BlockSpec((1, ROWB, F1H), lambda i, c: (0, i, 0)),
            pl.BlockSpec((1, ROWB, F1H), lambda i, c: (1, i, 0)),
            pl.BlockSpec((ROWB, 1), lambda i, c: (i, 0)),
            pl.BlockSpec((1, 1, F1H, F2H), lambda i, c: (c, 0, 0, 0)),
            pl.BlockSpec((1, 1, F1H, F2H), lambda i, c: (c, 1, 0, 0)),
        ],
        out_specs=pl.BlockSpec((1, ROWB, F2H), lambda i, c: (c, i, 0)),
        out_shape=jax.ShapeDtypeStruct((NC, N, F2H), jnp.float32),
    )(acc, acc, h1p, h1p, dis, W1s, W1s)


def _final(acc, h2p, dis):
    def body(a0_ref, a1_ref, h0_ref, h1_ref, dis_ref, out_ref):
        dis = dis_ref[...]
        v0 = jnp.maximum(dis * (a0_ref[0, :, :] + h0_ref[0, :, :]), 0.0)
        v1 = jnp.maximum(dis * (a1_ref[0, :, :] + h1_ref[0, :, :]), 0.0)
        out_ref[:, :F2H] = v0
        out_ref[:, F2H:] = v1[:, :F2 - F2H]

    return pl.pallas_call(
        body,
        grid=(GRID,),
        in_specs=[
            pl.BlockSpec((1, ROWB, F2H), lambda i: (0, i, 0)),
            pl.BlockSpec((1, ROWB, F2H), lambda i: (1, i, 0)),
            pl.BlockSpec((1, ROWB, F2H), lambda i: (0, i, 0)),
            pl.BlockSpec((1, ROWB, F2H), lambda i: (1, i, 0)),
            pl.BlockSpec((ROWB, 1), lambda i: (i, 0)),
        ],
        out_specs=pl.BlockSpec((ROWB, F2), lambda i: (i, 0)),
        out_shape=jax.ShapeDtypeStruct((N, F2), jnp.float32),
    )(acc, acc, h2p, h2p, dis)


def kernel(x, edge_index, edge_weight, W0, W1):
    src = edge_index[0].astype(jnp.int32)
    dst = edge_index[1].astype(jnp.int32)
    w = edge_weight.astype(jnp.float32)

    pad = EPAD - src.shape[0]
    src_p = jnp.pad(src, (0, pad)).reshape(NS, NB, EB)
    dst_p = jnp.pad(dst, (0, pad)).reshape(NS, NB, EB)
    w_p = jnp.pad(w, (0, pad)).reshape(NS, NB, EB)
    W1s = (jnp.pad(W1, ((0, 0), (0, F2P - F2)))
           .reshape(NC, F1H, NC, F2H).transpose(2, 0, 1, 3))

    deg_p = _deg_kernel(dst_p, w_p).reshape(1, NC, NPAD, 1)

    W0s = W0.reshape(D_IN, NC, F1H).transpose(1, 0, 2)
    dis, h1p = _layer1(x, W0s, deg_p)
    acc1 = _agg_l1(h1p, src_p, dst_p, w_p)
    h2p = _layer2(acc1, h1p, dis, W1s)
    acc2 = _agg_l2(h2p, src_p, dst_p, w_p)
    return _final(acc2, h2p, dis)


# speedup vs baseline: 37.9415x; 1.1077x over previous
import functools

import jax
import jax.numpy as jnp
from jax import lax
from jax.experimental import pallas as pl
from jax.experimental.pallas import tpu as pltpu
from jax.experimental.pallas import tpu_sc as plsc

N = 10000
D_IN = 128
F1 = 48
F1H = F1 // 2
F2 = 60
F2P = 64
F2H = F2P // 2

NC = 2
NS = 16

NPAD = 10240
RPS = NPAD // NS
RDIS = NPAD // (NC * NS)

EB = 128
NB = 160
EPAD = NS * NB * EB
NSLOT = 8

ROWB = 2048
GRID = NPAD // ROWB

_MESH = plsc.VectorSubcoreMesh(core_axis_name="c", subcore_axis_name="s")


def _vslices(fph):
    starts = list(range(0, fph - 15, 16))
    if fph % 16:
        starts.append(fph - 16)
    return starts


def _rsqrt16(d):
    xi = lax.bitcast_convert_type(d, jnp.int32)
    yi = jnp.int32(0x5F3759DF) - (xi >> 1)
    y = lax.bitcast_convert_type(yi, jnp.float32)
    for _ in range(3):
        y = y * (1.5 - 0.5 * d * y * y)
    return y


@functools.partial(
    pl.kernel,
    out_type=jax.ShapeDtypeStruct((NPAD,), jnp.float32),
    mesh=_MESH,
    compiler_params=pltpu.CompilerParams(use_tc_tiling_on_sc=False, needs_layout_passes=False),
    scratch_types=[
        pltpu.VMEM((NB, NC, EB), jnp.int32),
        pltpu.VMEM((NB, EB), jnp.float32),
        pltpu.VMEM((RPS,), jnp.float32),
        pltpu.VMEM((RDIS,), jnp.float32),
        pltpu.VMEM_SHARED((NPAD,), jnp.float32),
    ],
)
def _dis_kernel(idx_hbm, w_hbm, dis_hbm, ev, w_v, zbuf, obuf, deg_sh):
    cid = lax.axis_index("c")
    sid = lax.axis_index("s")
    pltpu.sync_copy(idx_hbm.at[sid], ev)
    pltpu.sync_copy(w_hbm.at[sid], w_v)

    zero16 = jnp.zeros((16,), jnp.float32)

    @pl.loop(0, RPS // 16)
    def _(i):
        zbuf[pl.ds(i * 16, 16)] = zero16

    pltpu.sync_copy(zbuf, deg_sh.at[pl.ds(sid * RPS, RPS)])
    plsc.subcore_barrier()

    @pl.loop(0, NB)
    def _(j):
        pltpu.sync_copy(w_v.at[j], deg_sh.at[ev.at[j, 1]], add=True)

    plsc.subcore_barrier()

    off = (cid * NS + sid) * RDIS
    pltpu.sync_copy(deg_sh.at[pl.ds(off, RDIS)], obuf)

    @pl.loop(0, RDIS // 16)
    def _(g):
        d = obuf[pl.ds(g * 16, 16)] + 1.0
        obuf[pl.ds(g * 16, 16)] = _rsqrt16(d)

    pltpu.sync_copy(obuf, dis_hbm.at[pl.ds(off, RDIS)])


def _make_agg(fph, final):
    starts = _vslices(fph)
    out_shape = (jax.ShapeDtypeStruct((N, F2P), jnp.float32) if final
                 else jax.ShapeDtypeStruct((NC, NPAD, fph), jnp.float32))

    @functools.partial(
        pl.kernel,
        out_type=out_shape,
        mesh=_MESH,
        compiler_params=pltpu.CompilerParams(use_tc_tiling_on_sc=False, needs_layout_passes=False),
        scratch_types=[
            pltpu.VMEM((NSLOT, NC, EB), jnp.int32),
            pltpu.VMEM((NSLOT, EB), jnp.float32),
            pltpu.VMEM((NPAD,), jnp.float32),
            pltpu.VMEM((EB, fph), jnp.float32),
            pltpu.VMEM((EB, fph), jnp.float32),
            pltpu.VMEM((EB, fph), jnp.float32),
            pltpu.VMEM((EB, fph), jnp.float32),
            pltpu.VMEM_SHARED((NPAD, fph), jnp.float32),
            pltpu.VMEM_SHARED((NPAD, fph), jnp.float32),
            [pltpu.SemaphoreType.DMA] * NSLOT,
            pltpu.SemaphoreType.DMA,
            pltpu.SemaphoreType.DMA,
            pltpu.SemaphoreType.DMA,
            pltpu.SemaphoreType.DMA,
        ],
    )
    def agg_kernel(h_hbm, idx_hbm, w_hbm, dis_hbm, out_hbm,
                   ib, wb, dis_v, g0, g1, s0, s1, acc_sh, h_sh,
                   isems, gsem0, gsem1, ssem0, ssem1):
        cid = lax.axis_index("c")
        sid = lax.axis_index("s")

        pltpu.sync_copy(dis_hbm, dis_v)
        pltpu.sync_copy(h_hbm.at[cid, pl.ds(sid * RPS, RPS)],
                        h_sh.at[pl.ds(sid * RPS, RPS)])

        zero16 = jnp.zeros((16,), jnp.float32)

        @pl.loop(0, EB)
        def _(r):
            for st in starts:
                g0[r, pl.ds(st, 16)] = zero16

        @pl.loop(0, RPS // EB)
        def _(i):
            pltpu.sync_copy(g0, acc_sh.at[pl.ds(sid * RPS + i * EB, EB)])

        plsc.subcore_barrier()

        gbufs = (g0, g1)
        sbufs = (s0, s1)
        gsems = (gsem0, gsem1)
        ssems = (ssem0, ssem1)

        for n in range(NSLOT - 2):
            pltpu.async_copy(idx_hbm.at[sid, n], ib.at[n], isems[n])
            pltpu.async_copy(w_hbm.at[sid, n], wb.at[n], isems[n])
        for n in range(2):
            pltpu.make_async_copy(idx_hbm.at[sid, n], ib.at[n], isems[n]).wait()
            pltpu.make_async_copy(w_hbm.at[sid, n], wb.at[n], isems[n]).wait()
            pltpu.async_copy(h_sh.at[ib.at[n, 0]], gbufs[n], gsems[n])

        @pl.loop(0, NB, step=NSLOT)
        def _(j0):
            for b in range(NSLOT):
                j = j0 + b
                g, s = gbufs[b % 2], sbufs[b % 2]
                pltpu.make_async_copy(
                    h_sh.at[ib.at[b, 0]], g, gsems[b % 2]).wait()

                @pl.when(j >= 2)
                def _(s=s, b=b):
                    pltpu.make_async_copy(
                        s, acc_sh.at[ib.at[(b - 2) % NSLOT, 1]],
                        ssems[b % 2]).wait()

                @pl.when(j + NSLOT - 2 < NB)
                def _(b=b, j=j):
                    sl = (b + NSLOT - 2) % NSLOT
                    pltpu.async_copy(idx_hbm.at[sid, j + NSLOT - 2],
                                     ib.at[sl], isems[sl])
                    pltpu.async_copy(w_hbm.at[sid, j + NSLOT - 2],
                                     wb.at[sl], isems[sl])

                @plsc.parallel_loop(0, EB // 16, unroll=4)
                def _(gi, g=g, s=s, b=b):
                    src16 = ib[b, 0, pl.ds(gi * 16, 16)]
                    f16 = (wb[b, pl.ds(gi * 16, 16)]
                           * plsc.load_gather(dis_v, [src16]))
                    for i in range(16):
                        fv = jnp.full((16,), f16[i])
                        r = gi * 16 + i
                        for st in starts:
                            s[r, pl.ds(st, 16)] = g[r, pl.ds(st, 16)] * fv

                @pl.when(j + 2 < NB)
                def _(g=g, b=b, j=j):
                    sl = (b + 2) % NSLOT
                    pltpu.make_async_copy(idx_hbm.at[sid, j + 2],
                                          ib.at[sl], isems[sl]).wait()
                    pltpu.make_async_copy(w_hbm.at[sid, j + 2],
                                          wb.at[sl], isems[sl]).wait()
                    pltpu.async_copy(h_sh.at[ib.at[sl, 0]], g, gsems[b % 2])

                pltpu.async_copy(s, acc_sh.at[ib.at[b, 1]],
                                 ssems[b % 2], add=True)

        for b in (NSLOT - 2, NSLOT - 1):
            pltpu.make_async_copy(
                sbufs[b % 2], acc_sh.at[ib.at[b, 1]], ssems[b % 2]).wait()

        plsc.subcore_barrier()

        @pl.loop(0, RPS // EB)
        def _(ci):
            row0 = sid * RPS + ci * EB
            pltpu.sync_copy(h_sh.at[pl.ds(row0, EB)], g0)
            pltpu.sync_copy(acc_sh.at[pl.ds(row0, EB)], g1)

            @plsc.parallel_loop(0, EB // 16, unroll=2)
            def _(gi, row0=row0):
                d16 = dis_v[pl.ds(row0 + gi * 16, 16)]
                for i in range(16):
                    dv = jnp.full((16,), d16[i])
                    r = gi * 16 + i
                    for st in starts:
                        s0[r, pl.ds(st, 16)] = jnp.maximum(
                            dv * (g1[r, pl.ds(st, 16)]
                                  + dv * g0[r, pl.ds(st, 16)]), 0.0)

            if not final:
                pltpu.sync_copy(s0, out_hbm.at[cid, pl.ds(row0, EB)])
            else:
                @pl.when(row0 + EB <= N)
                def _(row0=row0):
                    pltpu.sync_copy(
                        s0, out_hbm.at[pl.ds(row0, EB),
                                       pl.ds(cid * F2H, F2H)])

                @pl.when(jnp.logical_and(row0 < N, row0 + EB > N))
                def _(row0=row0):
                    tail = N % EB
                    pltpu.sync_copy(
                        s0.at[pl.ds(0, tail)],
                        out_hbm.at[pl.ds(row0, tail), pl.ds(cid * F2H, F2H)])

    return agg_kernel


_agg_l1 = _make_agg(F1H, final=False)
_agg_l2 = _make_agg(F2H, final=True)


def _mm1(x, W0s):
    def body(x_ref, w_ref, h_ref):
        h_ref[0, :, :] = jnp.dot(x_ref[...], w_ref[0],
                                 preferred_element_type=jnp.float32)

    return pl.pallas_call(
        body,
        grid=(GRID, NC),
        in_specs=[
            pl.BlockSpec((ROWB, D_IN), lambda i, c: (i, 0)),
            pl.BlockSpec((1, D_IN, F1H), lambda i, c: (c, 0, 0)),
        ],
        out_specs=pl.BlockSpec((1, ROWB, F1H), lambda i, c: (c, i, 0)),
        out_shape=jax.ShapeDtypeStruct((NC, NPAD, F1H), jnp.float32),
    )(x, W0s)


def _mm2(t, W1s):
    def body(t0_ref, t1_ref, w0_ref, w1_ref, out_ref):
        out_ref[0, :, :] = (
            jnp.dot(t0_ref[0], w0_ref[0, 0], preferred_element_type=jnp.float32)
            + jnp.dot(t1_ref[0], w1_ref[0, 0],
                      preferred_element_type=jnp.float32))

    return pl.pallas_call(
        body,
        grid=(GRID, NC),
        in_specs=[
            pl.BlockSpec((1, ROWB, F1H), lambda i, c: (0, i, 0)),
            pl.BlockSpec((1, ROWB, F1H), lambda i, c: (1, i, 0)),
            pl.BlockSpec((1, 1, F1H, F2H), lambda i, c: (c, 0, 0, 0)),
            pl.BlockSpec((1, 1, F1H, F2H), lambda i, c: (c, 1, 0, 0)),
        ],
        out_specs=pl.BlockSpec((1, ROWB, F2H), lambda i, c: (c, i, 0)),
        out_shape=jax.ShapeDtypeStruct((NC, NPAD, F2H), jnp.float32),
    )(t, t, W1s, W1s)


def kernel(x, edge_index, edge_weight, W0, W1):
    src = edge_index[0].astype(jnp.int32)
    dst = edge_index[1].astype(jnp.int32)
    w = edge_weight.astype(jnp.float32)

    pad = EPAD - src.shape[0]
    src_p = jnp.pad(src, (0, pad)).reshape(NS, NB, EB)
    dst_p = jnp.pad(dst, (0, pad)).reshape(NS, NB, EB)
    idx_p = jnp.stack([src_p, dst_p], axis=2)
    w_p = jnp.pad(w, (0, pad)).reshape(NS, NB, EB)

    W0s = W0.reshape(D_IN, NC, F1H).transpose(1, 0, 2)
    W1s = (jnp.pad(W1, ((0, 0), (0, F2P - F2)))
           .reshape(NC, F1H, NC, F2H).transpose(2, 0, 1, 3))

    dis = _dis_kernel(idx_p, w_p)
    m1 = _mm1(x, W0s)
    t = _agg_l1(m1, idx_p, w_p, dis)
    h2 = _mm2(t, W1s)
    return _agg_l2(h2, idx_p, w_p, dis)[:, :F2]
